# baseline (speedup 1.0000x reference)
import functools

import jax
import jax.numpy as jnp
from jax import lax
from jax.experimental import pallas as pl
from jax.experimental.pallas import tpu as pltpu
from jax.experimental.pallas import tpu_sc as plsc

_EMB_DIM = 64
_NC = 2
_NS = 16
_NW = _NC * _NS

_G = 128
_CHUNK = 512
_GPC = _CHUNK // _G


def _make_gather(B: int, D: int):
    b_per_w = B // _NW
    n_chunks = b_per_w // _CHUNK
    idx_rows_per_w = b_per_w // _G

    mesh = plsc.VectorSubcoreMesh(core_axis_name="c", subcore_axis_name="s")

    @functools.partial(
        pl.kernel,
        mesh=mesh,
        out_type=jax.ShapeDtypeStruct((B, D), jnp.float32),
        compiler_params=pltpu.CompilerParams(use_tc_tiling_on_sc=False),
        scratch_types=[
            pltpu.VMEM((idx_rows_per_w, _G), jnp.int32),
            pltpu.VMEM((_CHUNK, D), jnp.float32),
            pltpu.SemaphoreType.DMA,
        ],
    )
    def gather(table_hbm, idx_hbm, out_hbm, idx_v, rows_v, sem):
        wid = lax.axis_index("s") * _NC + lax.axis_index("c")
        base = wid * b_per_w
        pltpu.sync_copy(idx_hbm.at[pl.ds(wid * idx_rows_per_w, idx_rows_per_w)],
                        idx_v)

        def chunk_body(c, carry):
            copies = []
            for j in range(_GPC):
                copies.append(pltpu.async_copy(
                    table_hbm.at[idx_v.at[c * _GPC + j]],
                    rows_v.at[pl.ds(j * _G, _G)],
                    sem))
            for cp in copies:
                cp.wait()
            pltpu.sync_copy(rows_v, out_hbm.at[pl.ds(base + c * _CHUNK, _CHUNK)])
            return carry

        lax.fori_loop(0, n_chunks, chunk_body, 0)

    return gather


def kernel(batch_ids, attention_mask, emb_weight):
    batch, seq = batch_ids.shape
    b = batch * seq
    idx2d = batch_ids.reshape(b // _G, _G)
    out = _make_gather(b, _EMB_DIM)(emb_weight, idx2d)
    return out.reshape(batch, seq, _EMB_DIM), attention_mask


# speedup vs baseline: 1.6254x; 1.6254x over previous
import functools

import jax
import jax.numpy as jnp
from jax import lax
from jax.experimental import pallas as pl
from jax.experimental.pallas import tpu as pltpu
from jax.experimental.pallas import tpu_sc as plsc

_EMB_DIM = 64
_NC = 2
_NS = 16
_NW = _NC * _NS

_G = 128
_CHUNK = 512
_GPC = _CHUNK // _G
_TBLK = 2048


def _transpose_table(tt):
    d, v = tt.shape
    half = _TBLK // 2
    nblk = (v + _TBLK - 1) // _TBLK

    def body(in_ref, out_ref):
        xt = in_ref[...].T
        out_ref[...] = jnp.concatenate([xt[:half], xt[half:]], axis=1)

    return pl.pallas_call(
        body,
        grid=(nblk,),
        in_specs=[pl.BlockSpec((d, _TBLK), lambda i: (0, i))],
        out_specs=pl.BlockSpec((half, 128), lambda i: (i, 0)),
        out_shape=jax.ShapeDtypeStruct((nblk * half, 128), jnp.float32),
    )(tt)


def _transpose_out(rows2, batch, seq):
    nb = 128
    rpb = nb * seq // 2

    def body(in_ref, out_ref):
        x = in_ref[...]
        x3 = x.reshape(nb, seq // 2, 128)
        x3 = x3.transpose(1, 0, 2)
        c3 = x3.transpose(0, 2, 1)
        out_ref[...] = c3.reshape(seq, 64, nb)

    return pl.pallas_call(
        body,
        grid=(batch // nb,),
        in_specs=[pl.BlockSpec((rpb, 128), lambda i: (i, 0))],
        out_specs=pl.BlockSpec((seq, 64, nb), lambda i: (0, 0, i)),
        out_shape=jax.ShapeDtypeStruct((seq, 64, batch), jnp.float32),
    )(rows2)


def _make_gather(B: int, D: int):
    b_per_w = B // _NW
    n_chunks = b_per_w // _CHUNK
    idx_rows_per_w = b_per_w // _G
    vregs_per_row = _G // 16

    mesh = plsc.VectorSubcoreMesh(core_axis_name="c", subcore_axis_name="s")

    @functools.partial(
        pl.kernel,
        mesh=mesh,
        out_type=jax.ShapeDtypeStruct((B, D), jnp.float32),
        scratch_types=[
            pltpu.VMEM((idx_rows_per_w, _G), jnp.int32),
            pltpu.VMEM((_CHUNK, D), jnp.float32),
            pltpu.SemaphoreType.DMA,
        ],
        compiler_params=pltpu.CompilerParams(
            use_tc_tiling_on_sc=False, needs_layout_passes=False),
    )
    def gather(table_hbm, idx_hbm, out_hbm, idx_v, rows_v, sem):
        wid = lax.axis_index("s") * _NC + lax.axis_index("c")
        base = wid * b_per_w
        pltpu.sync_copy(idx_hbm.at[pl.ds(wid * idx_rows_per_w, idx_rows_per_w)],
                        idx_v)

        def remap_body(r, carry):
            for j in range(vregs_per_row):
                t = idx_v[r, pl.ds(j * 16, 16)]
                k = ((t & jnp.int32(~2047))
                     + ((t & jnp.int32(1023)) << 1)
                     + ((t >> 10) & jnp.int32(1)))
                idx_v[r, pl.ds(j * 16, 16)] = k
            return carry

        lax.fori_loop(0, idx_rows_per_w, remap_body, 0)

        def chunk_body(c, carry):
            copies = []
            for j in range(_GPC):
                copies.append(pltpu.async_copy(
                    table_hbm.at[idx_v.at[c * _GPC + j]],
                    rows_v.at[pl.ds(j * _G, _G)],
                    sem))
            for cp in copies:
                cp.wait()
            pltpu.sync_copy(rows_v, out_hbm.at[pl.ds(base + c * _CHUNK, _CHUNK)])
            return carry

        lax.fori_loop(0, n_chunks, chunk_body, 0)

    return gather


def kernel(batch_ids, attention_mask, emb_weight):
    batch, seq = batch_ids.shape
    vocab, d = emb_weight.shape
    b = batch * seq

    table2 = _transpose_table(emb_weight.T)
    table = table2.reshape(-1, d)
    idx2d = batch_ids.reshape(b // _G, _G)
    out = _make_gather(b, d)(table, idx2d)
    out_t = _transpose_out(out.reshape(b // 2, 128), batch, seq)
    return out_t.transpose(2, 0, 1), attention_mask


# speedup vs baseline: 1.6379x; 1.0077x over previous
import functools

import jax
import jax.numpy as jnp
from jax import lax
from jax.experimental import pallas as pl
from jax.experimental.pallas import tpu as pltpu
from jax.experimental.pallas import tpu_sc as plsc

_EMB_DIM = 64
_NC = 2
_NS = 16
_NW = _NC * _NS

_G = 128
_CHUNK = 512
_GPC = _CHUNK // _G
_TBLK = 2048


def _transpose_table(tt):
    d, v = tt.shape
    half = _TBLK // 2
    nblk = (v + _TBLK - 1) // _TBLK

    def body(in_ref, out_ref):
        xt = in_ref[...].T
        out_ref[...] = jnp.concatenate([xt[:half], xt[half:]], axis=1)

    return pl.pallas_call(
        body,
        grid=(nblk,),
        in_specs=[pl.BlockSpec((d, _TBLK), lambda i: (0, i))],
        out_specs=pl.BlockSpec((half, 128), lambda i: (i, 0)),
        out_shape=jax.ShapeDtypeStruct((nblk * half, 128), jnp.float32),
    )(tt)


def _transpose_out(rows2, batch, seq_p, part, seq, prev):
    nb = 128
    rpb = nb * seq_p // 2

    def body(in_ref, *rest):
        out_ref = rest[-1]
        x = in_ref[...]
        x3 = x.reshape(nb, seq_p // 2, 128)
        x3 = x3.transpose(1, 0, 2)
        c3 = x3.transpose(0, 2, 1)
        out_ref[...] = c3.reshape(seq_p, 64, nb)

    in_specs = [pl.BlockSpec((rpb, 128), lambda i: (i, 0))]
    operands = [rows2]
    aliases = {}
    if prev is not None:
        in_specs.append(pl.BlockSpec(memory_space=pl.ANY))
        operands.append(prev)
        aliases = {1: 0}

    return pl.pallas_call(
        body,
        grid=(batch // nb,),
        in_specs=in_specs,
        out_specs=pl.BlockSpec((seq_p, 64, nb), lambda i: (part, 0, i)),
        out_shape=jax.ShapeDtypeStruct((seq, 64, batch), jnp.float32),
        input_output_aliases=aliases,
    )(*operands)


def _make_gather(B: int, D: int):
    b_per_w = B // _NW
    n_chunks = b_per_w // _CHUNK
    idx_rows_per_w = b_per_w // _G
    vregs_per_row = _G // 16

    mesh = plsc.VectorSubcoreMesh(core_axis_name="c", subcore_axis_name="s")

    @functools.partial(
        pl.kernel,
        mesh=mesh,
        out_type=jax.ShapeDtypeStruct((B, D), jnp.float32),
        scratch_types=[
            pltpu.VMEM((idx_rows_per_w, _G), jnp.int32),
            pltpu.VMEM((_CHUNK, D), jnp.float32),
            pltpu.SemaphoreType.DMA,
        ],
        compiler_params=pltpu.CompilerParams(
            use_tc_tiling_on_sc=False, needs_layout_passes=False),
    )
    def gather(table_hbm, idx_hbm, out_hbm, idx_v, rows_v, sem):
        wid = lax.axis_index("s") * _NC + lax.axis_index("c")
        base = wid * b_per_w
        pltpu.sync_copy(idx_hbm.at[pl.ds(wid * idx_rows_per_w, idx_rows_per_w)],
                        idx_v)

        def remap_body(r, carry):
            for j in range(vregs_per_row):
                t = idx_v[r, pl.ds(j * 16, 16)]
                k = ((t & jnp.int32(~2047))
                     + ((t & jnp.int32(1023)) << 1)
                     + ((t >> 10) & jnp.int32(1)))
                idx_v[r, pl.ds(j * 16, 16)] = k
            return carry

        lax.fori_loop(0, idx_rows_per_w, remap_body, 0)

        def chunk_body(c, carry):
            copies = []
            for j in range(_GPC):
                copies.append(pltpu.async_copy(
                    table_hbm.at[idx_v.at[c * _GPC + j]],
                    rows_v.at[pl.ds(j * _G, _G)],
                    sem))
            for cp in copies:
                cp.wait()
            pltpu.sync_copy(rows_v, out_hbm.at[pl.ds(base + c * _CHUNK, _CHUNK)])
            return carry

        lax.fori_loop(0, n_chunks, chunk_body, 0)

    return gather


_NPARTS = 2


def kernel(batch_ids, attention_mask, emb_weight):
    batch, seq = batch_ids.shape
    vocab, d = emb_weight.shape
    seq_p = seq // _NPARTS
    b_p = batch * seq_p

    table2 = _transpose_table(emb_weight.T)
    table = table2.reshape(-1, d)
    gather = _make_gather(b_p, d)
    out_t = None
    for p in range(_NPARTS):
        idx_p = batch_ids[:, p * seq_p:(p + 1) * seq_p].reshape(b_p // _G, _G)
        rows = gather(table, idx_p)
        out_t = _transpose_out(rows.reshape(b_p // 2, 128),
                               batch, seq_p, p, seq, out_t)
    return out_t.transpose(2, 0, 1), attention_mask
